# Initial kernel scaffold; baseline (speedup 1.0000x reference)
import jax
import jax.numpy as jnp
from jax.experimental import pallas as pl


def kernel(coords_r_theta, coords_phi, poses, fmap1, i, j):
    raise NotImplementedError("write your pallas kernel here")


# speedup vs baseline: 3.1501x; 3.1501x over previous
import jax
import jax.numpy as jnp
from jax.experimental import pallas as pl
from jax.experimental.pallas import tpu as pltpu

R_MIN = 0.5
R_MAX = 30.0
FLS_H = 256
FLS_W = 128
FOV_H = 2.0944
FOV_V = 0.3491
PITCH = -0.1745
PHI_MAX = -PITCH
PHI_MIN = -PITCH - FOV_V
DS = 4
C = 64
COORDS_EPS = 0.05
PAD = 9
H2 = FLS_H // DS
W2 = FLS_W // DS
H1P, W1P = FLS_H + 2 * PAD, FLS_W + 2 * PAD
H2P, W2P = H2 + 2 * PAD, W2 + 2 * PAD


def _quat_rotate(q, v, inverse=False):
    q = q / (jnp.linalg.norm(q, axis=-1, keepdims=True) + 1e-12)
    w = q[:, :1]
    u = q[:, 1:]
    if inverse:
        u = -u
    uv = jnp.cross(u, v)
    return v + 2.0 * jnp.cross(u, uv + w * v)


def _pool_lead_kernel(x_ref, o_ref):
    x = x_ref[0]
    k = x.shape[0] // DS
    o_ref[0] = x.reshape(k, DS, x.shape[1], C).sum(axis=1) * (1.0 / DS)


def _sample_kernel(js_ref, od_ref, ty_ref, tx_ref, vl_ref,
                   f1_ref, f2_ref, o1_ref, o2_ref):
    o = od_ref[pl.program_id(0)]
    ty1 = ty_ref[o]
    tx1 = tx_ref[o]
    ty2 = ty1 * (1.0 / DS)
    tx2 = tx1 * (1.0 / DS)
    vld = vl_ref[o]

    def patch(f_ref, ty, tx, hh, ww):
        fy = jnp.floor(ty)
        fx = jnp.floor(tx)
        wy = ty - fy
        wx = tx - fx
        sy = jnp.clip(fy, -5.0, hh + 3.0).astype(jnp.int32) - 4 + PAD
        sx = jnp.clip(fx, -5.0, ww + 3.0).astype(jnp.int32) - 4 + PAD
        win = f_ref[0, pl.ds(sy, 10), pl.ds(sx, 10), :]
        ry = jax.lax.broadcasted_iota(jnp.int32, (9, 9), 0).astype(jnp.float32) + (fy - 4.0)
        rx = jax.lax.broadcasted_iota(jnp.int32, (9, 9), 1).astype(jnp.float32) + (fx - 4.0)

        def tap(a, b):
            yy = ry + a
            xx = rx + b
            m = ((yy >= 0) & (yy <= hh - 1) & (xx >= 0) & (xx <= ww - 1))
            wgt = (wy if a else (1.0 - wy)) * (wx if b else (1.0 - wx))
            return win[a:a + 9, b:b + 9, :] * (m.astype(jnp.float32) * wgt)[:, :, None]

        return (tap(0, 0) + tap(0, 1) + tap(1, 0) + tap(1, 1)) * vld

    o1_ref[0] = patch(f1_ref, ty1, tx1, FLS_H, FLS_W)
    o2_ref[0] = patch(f2_ref, ty2, tx2, H2, W2)


def kernel(coords_r_theta, coords_phi, poses, fmap1, i, j):
    b, n, p_, _ = coords_r_theta.shape
    bn = b * n
    e = i.shape[0]

    poses_flat = poses.reshape(bn, 7)
    src_poses = poses_flat[i // p_]
    tgt_poses = poses_flat[j]
    crt = coords_r_theta.reshape(bn * p_, 2)
    cph = coords_phi.reshape(bn * p_, 1)
    src_coords = jnp.concatenate([crt, cph], axis=1)[i]
    r = src_coords[:, 0]
    th = src_coords[:, 1]
    ph = src_coords[:, 2]
    pt = jnp.stack([r * jnp.cos(ph) * jnp.cos(th),
                    r * jnp.cos(ph) * jnp.sin(th),
                    r * jnp.sin(ph)], axis=-1)
    world = _quat_rotate(src_poses[:, 3:7], pt) + src_poses[:, :3]
    loc = _quat_rotate(tgt_poses[:, 3:7], world - tgt_poses[:, :3], inverse=True)
    r2 = jnp.linalg.norm(loc, axis=-1)
    th2 = jnp.arctan2(loc[:, 1], loc[:, 0])
    ph2 = jnp.arcsin(jnp.clip(loc[:, 2] / (r2 + 1e-9), -0.999999, 0.999999))
    theta_max = FOV_H / 2.0
    oor = (r2 < R_MIN - COORDS_EPS) | (r2 > R_MAX + COORDS_EPS)
    oor = oor | (jnp.abs(th2) > theta_max + COORDS_EPS)
    oor = oor | (ph2 > PHI_MAX + COORDS_EPS) | (ph2 < PHI_MIN - COORDS_EPS)
    valid = (~oor).astype(fmap1.dtype)
    ty1 = (r2 - R_MIN) / (R_MAX - R_MIN) * (FLS_H - 1)
    tx1 = (th2 / FOV_H + 0.5) * (FLS_W - 1)

    fmap_hwc = fmap1.reshape(bn, C, FLS_H, FLS_W).transpose(0, 2, 3, 1)

    fy2 = pl.pallas_call(
        _pool_lead_kernel,
        grid=(bn,),
        in_specs=[pl.BlockSpec((1, FLS_H, FLS_W, C), lambda f: (f, 0, 0, 0))],
        out_specs=pl.BlockSpec((1, H2, FLS_W, C), lambda f: (f, 0, 0, 0)),
        out_shape=jax.ShapeDtypeStruct((bn, H2, FLS_W, C), jnp.float32),
    )(fmap_hwc)
    fy2t = fy2.transpose(0, 2, 1, 3)
    f2t = pl.pallas_call(
        _pool_lead_kernel,
        grid=(bn,),
        in_specs=[pl.BlockSpec((1, FLS_W, H2, C), lambda f: (f, 0, 0, 0))],
        out_specs=pl.BlockSpec((1, W2, H2, C), lambda f: (f, 0, 0, 0)),
        out_shape=jax.ShapeDtypeStruct((bn, W2, H2, C), jnp.float32),
    )(fy2t)
    f2 = f2t.transpose(0, 2, 1, 3)

    f1p = jnp.pad(fmap_hwc, ((0, 0), (PAD, PAD), (PAD, PAD), (0, 0)))
    f2p = jnp.pad(f2, ((0, 0), (PAD, PAD), (PAD, PAD), (0, 0)))

    order = jnp.argsort(j).astype(jnp.int32)
    j_sorted = j[order].astype(jnp.int32)

    grid_spec = pltpu.PrefetchScalarGridSpec(
        num_scalar_prefetch=5,
        grid=(e,),
        in_specs=[
            pl.BlockSpec((1, H1P, W1P, C), lambda ei, *s: (s[0][ei], 0, 0, 0)),
            pl.BlockSpec((1, H2P, W2P, C), lambda ei, *s: (s[0][ei], 0, 0, 0)),
        ],
        out_specs=[
            pl.BlockSpec((1, 9, 9, C), lambda ei, *s: (s[1][ei], 0, 0, 0)),
            pl.BlockSpec((1, 9, 9, C), lambda ei, *s: (s[1][ei], 0, 0, 0)),
        ],
    )
    p1, p2 = pl.pallas_call(
        _sample_kernel,
        grid_spec=grid_spec,
        out_shape=[jax.ShapeDtypeStruct((e, 9, 9, C), jnp.float32),
                   jax.ShapeDtypeStruct((e, 9, 9, C), jnp.float32)],
    )(j_sorted, order, ty1, tx1, valid, f1p, f2p)

    return p1.transpose(0, 3, 1, 2), p2.transpose(0, 3, 1, 2), valid
